# Initial kernel scaffold; baseline (speedup 1.0000x reference)
import jax
import jax.numpy as jnp
from jax.experimental import pallas as pl


def kernel(indices, query_wemb, centroids_k):
    raise NotImplementedError("write your pallas kernel here")


# speedup vs baseline: 1.5343x; 1.5343x over previous
import functools

import jax
import jax.numpy as jnp
from jax import lax
from jax.experimental import pallas as pl
from jax.experimental.pallas import tpu as pltpu
from jax.experimental.pallas import tpu_sc as plsc

K = 512
D = 8
EMB = 64
SUB = EMB // D
DK = D * K

HIGH = lax.Precision.HIGHEST


def _make_sc_gather(n_rows: int):
    info = plsc.get_sparse_core_info()
    nw = info.num_cores * info.num_subcores
    ch = 128
    assert n_rows % (nw * ch) == 0
    nch = n_rows // (nw * ch)
    mesh = plsc.VectorSubcoreMesh(core_axis_name="c", subcore_axis_name="s")

    @functools.partial(
        pl.kernel,
        mesh=mesh,
        out_type=jax.ShapeDtypeStruct((n_rows, EMB), jnp.float32),
        scratch_types=[
            pltpu.VMEM((nch, ch), jnp.int32),
            pltpu.VMEM((ch, EMB), jnp.float32),
            pltpu.VMEM((ch, EMB), jnp.float32),
            pltpu.SemaphoreType.DMA,
            pltpu.SemaphoreType.DMA,
        ],
        compiler_params=pltpu.CompilerParams(use_tc_tiling_on_sc=False),
    )
    def sc_gather(table_hbm, idx_hbm, out_hbm, idx_v, buf0, buf1, sem0, sem1):
        wid = lax.axis_index("s") * info.num_cores + lax.axis_index("c")
        row0 = wid * nch
        pltpu.sync_copy(idx_hbm.at[wid], idx_v)
        bufs = (buf0, buf1)
        sems = (sem0, sem1)
        pltpu.async_copy(table_hbm.at[idx_v.at[0]], bufs[0], sems[0])
        for c in range(nch):
            if c + 1 < nch:
                pltpu.async_copy(
                    table_hbm.at[idx_v.at[c + 1]], bufs[(c + 1) % 2],
                    sems[(c + 1) % 2])
            pltpu.make_async_copy(
                table_hbm.at[idx_v.at[c]], bufs[c % 2], sems[c % 2]).wait()
            pltpu.sync_copy(bufs[c % 2],
                            out_hbm.at[pl.ds((row0 + c) * ch, ch)])

    return sc_gather


def _stats_kernel(x_ref, w_ref, sum_ref, sq_ref):
    @pl.when(pl.program_id(0) == 0)
    def _init():
        sum_ref[...] = jnp.zeros_like(sum_ref)
        sq_ref[...] = jnp.zeros_like(sq_ref)

    x = x_ref[...]
    dot = lax.dot_general(x, w_ref[...], (((1,), (0,)), ((), ())),
                          preferred_element_type=jnp.float32)
    c2 = jnp.sum(w_ref[...] * w_ref[...], axis=0, keepdims=True)
    xsq = x * x
    s_acc = jnp.zeros((1, K), jnp.float32)
    q_acc = jnp.zeros((1, K), jnp.float32)
    for d in range(D):
        a = jnp.sum(xsq[:, d * SUB:(d + 1) * SUB], axis=1, keepdims=True)
        r = (-a + 2.0 * dot[:, d * K:(d + 1) * K]) - c2[:, d * K:(d + 1) * K]
        s_acc = s_acc + jnp.sum(r, axis=0, keepdims=True)
        q_acc = q_acc + jnp.sum(r * r, axis=0, keepdims=True)
    sum_ref[...] += s_acc
    sq_ref[...] += q_acc


def _argmax_kernel(n_total, x_ref, w_ref, wt_ref, sum_ref, sq_ref, out_ref):
    nd = jnp.float32(n_total * D)
    mean = sum_ref[...] / nd
    var = sq_ref[...] / nd - (sum_ref[...] / nd) * (sum_ref[...] / nd)
    denom = jnp.sqrt(var + 1e-3)
    x = x_ref[...]
    bn = x.shape[0]
    dot = lax.dot_general(x, w_ref[...], (((1,), (0,)), ((), ())),
                          preferred_element_type=jnp.float32)
    c2 = jnp.sum(w_ref[...] * w_ref[...], axis=0, keepdims=True)
    xsq = x * x
    iota = lax.broadcasted_iota(jnp.int32, (bn, K), 1)
    ohs = []
    for d in range(D):
        a = jnp.sum(xsq[:, d * SUB:(d + 1) * SUB], axis=1, keepdims=True)
        r = (-a + 2.0 * dot[:, d * K:(d + 1) * K]) - c2[:, d * K:(d + 1) * K]
        rn = (r - mean) / denom
        m = jnp.max(rn, axis=1, keepdims=True)
        code = jnp.min(jnp.where(rn == m, iota, K), axis=1, keepdims=True)
        ohs.append((iota == code).astype(jnp.float32))
    onehot = jnp.concatenate(ohs, axis=1)
    out_ref[...] = lax.dot_general(onehot, wt_ref[...],
                                   (((1,), (0,)), ((), ())),
                                   preferred_element_type=jnp.float32,
                                   precision=HIGH)


def kernel(indices, query_wemb, centroids_k):
    b, h = indices.shape
    n = b * h
    idx = indices.reshape(32, n // (32 * 128), 128).astype(jnp.int32)
    x = _make_sc_gather(n)(query_wemb, idx)

    ct = jnp.transpose(centroids_k, (0, 2, 1))
    dr = jnp.arange(D)
    w = (jnp.zeros((D, SUB, D, K), jnp.float32)
         .at[dr, :, dr, :].set(ct).reshape(EMB, DK))
    wt = (jnp.zeros((D, K, D, SUB), jnp.float32)
          .at[dr, :, dr, :].set(centroids_k).reshape(DK, EMB))

    bn = 512
    grid = (n // bn,)
    sums, sqs = pl.pallas_call(
        _stats_kernel,
        grid=grid,
        in_specs=[
            pl.BlockSpec((bn, EMB), lambda i: (i, 0)),
            pl.BlockSpec((EMB, DK), lambda i: (0, 0)),
        ],
        out_specs=[
            pl.BlockSpec((1, K), lambda i: (0, 0)),
            pl.BlockSpec((1, K), lambda i: (0, 0)),
        ],
        out_shape=[
            jax.ShapeDtypeStruct((1, K), jnp.float32),
            jax.ShapeDtypeStruct((1, K), jnp.float32),
        ],
    )(x, w)

    out = pl.pallas_call(
        functools.partial(_argmax_kernel, n),
        grid=grid,
        in_specs=[
            pl.BlockSpec((bn, EMB), lambda i: (i, 0)),
            pl.BlockSpec((EMB, DK), lambda i: (0, 0)),
            pl.BlockSpec((DK, EMB), lambda i: (0, 0)),
            pl.BlockSpec((1, K), lambda i: (0, 0)),
            pl.BlockSpec((1, K), lambda i: (0, 0)),
        ],
        out_specs=pl.BlockSpec((bn, EMB), lambda i: (i, 0)),
        out_shape=jax.ShapeDtypeStruct((n, EMB), jnp.float32),
    )(x, w, wt, sums, sqs)

    return out.reshape(b, h, EMB)


# speedup vs baseline: 1.7217x; 1.1221x over previous
import functools

import jax
import jax.numpy as jnp
from jax import lax
from jax.experimental import pallas as pl
from jax.experimental.pallas import tpu as pltpu
from jax.experimental.pallas import tpu_sc as plsc

K = 512
D = 8
EMB = 64
SUB = EMB // D
DK = D * K

HIGH = lax.Precision.HIGHEST


def _make_sc_gather(n_rows: int):
    info = plsc.get_sparse_core_info()
    nw = info.num_cores * info.num_subcores
    ch = 128
    assert n_rows % (nw * ch) == 0
    nch = n_rows // (nw * ch)
    mesh = plsc.VectorSubcoreMesh(core_axis_name="c", subcore_axis_name="s")

    @functools.partial(
        pl.kernel,
        mesh=mesh,
        out_type=jax.ShapeDtypeStruct((n_rows, EMB), jnp.float32),
        scratch_types=[
            pltpu.VMEM((nch, ch), jnp.int32),
            pltpu.VMEM((ch, EMB), jnp.float32),
            pltpu.VMEM((ch, EMB), jnp.float32),
            pltpu.SemaphoreType.DMA,
            pltpu.SemaphoreType.DMA,
        ],
        compiler_params=pltpu.CompilerParams(use_tc_tiling_on_sc=False),
    )
    def sc_gather(table_hbm, idx_hbm, out_hbm, idx_v, buf0, buf1, sem0, sem1):
        wid = lax.axis_index("s") * info.num_cores + lax.axis_index("c")
        row0 = wid * nch
        pltpu.sync_copy(idx_hbm.at[wid], idx_v)
        bufs = (buf0, buf1)
        sems = (sem0, sem1)
        pltpu.async_copy(table_hbm.at[idx_v.at[0]], bufs[0], sems[0])
        for c in range(nch):
            if c + 1 < nch:
                pltpu.async_copy(
                    table_hbm.at[idx_v.at[c + 1]], bufs[(c + 1) % 2],
                    sems[(c + 1) % 2])
            pltpu.make_async_copy(
                table_hbm.at[idx_v.at[c]], bufs[c % 2], sems[c % 2]).wait()
            pltpu.sync_copy(bufs[c % 2],
                            out_hbm.at[pl.ds((row0 + c) * ch, ch)])

    return sc_gather


def _stats_kernel(x_ref, w_ref, sum_ref, sq_ref):
    @pl.when(pl.program_id(0) == 0)
    def _init():
        sum_ref[...] = jnp.zeros_like(sum_ref)
        sq_ref[...] = jnp.zeros_like(sq_ref)

    x = x_ref[...]
    dot = lax.dot_general(x, w_ref[...], (((1,), (0,)), ((), ())),
                          preferred_element_type=jnp.float32)
    c2 = jnp.sum(w_ref[...] * w_ref[...], axis=0, keepdims=True)
    xsq = x * x
    s_acc = jnp.zeros((1, K), jnp.float32)
    q_acc = jnp.zeros((1, K), jnp.float32)
    for d in range(D):
        a = jnp.sum(xsq[:, d * SUB:(d + 1) * SUB], axis=1, keepdims=True)
        r = (-a + 2.0 * dot[:, d * K:(d + 1) * K]) - c2[:, d * K:(d + 1) * K]
        s_acc = s_acc + jnp.sum(r, axis=0, keepdims=True)
        q_acc = q_acc + jnp.sum(r * r, axis=0, keepdims=True)
    sum_ref[...] += s_acc
    sq_ref[...] += q_acc


def _argmax_kernel(n_total, x_ref, w_ref, wh_ref, wl_ref, sum_ref, sq_ref, out_ref):
    nd = jnp.float32(n_total * D)
    mean = sum_ref[...] / nd
    var = sq_ref[...] / nd - (sum_ref[...] / nd) * (sum_ref[...] / nd)
    inv = 1.0 / jnp.sqrt(var + 1e-3)
    x = x_ref[...]
    bn = x.shape[0]
    dot = lax.dot_general(x, w_ref[...], (((1,), (0,)), ((), ())),
                          preferred_element_type=jnp.float32)
    c2 = jnp.sum(w_ref[...] * w_ref[...], axis=0, keepdims=True)
    xsq = x * x
    iota = lax.broadcasted_iota(jnp.int32, (bn, K), 1)
    ohs = []
    for d in range(D):
        a = jnp.sum(xsq[:, d * SUB:(d + 1) * SUB], axis=1, keepdims=True)
        r = (-a + 2.0 * dot[:, d * K:(d + 1) * K]) - c2[:, d * K:(d + 1) * K]
        rn = (r - mean) * inv
        m = jnp.max(rn, axis=1, keepdims=True)
        code = jnp.min(jnp.where(rn == m, iota, K), axis=1, keepdims=True)
        ohs.append((iota == code).astype(jnp.float32))
    onehot = jnp.concatenate(ohs, axis=1)
    sel_hi = lax.dot_general(onehot, wh_ref[...].astype(jnp.float32),
                             (((1,), (0,)), ((), ())),
                             preferred_element_type=jnp.float32)
    sel_lo = lax.dot_general(onehot, wl_ref[...].astype(jnp.float32),
                             (((1,), (0,)), ((), ())),
                             preferred_element_type=jnp.float32)
    out_ref[...] = sel_hi + sel_lo


def kernel(indices, query_wemb, centroids_k):
    b, h = indices.shape
    n = b * h
    idx = indices.reshape(32, n // (32 * 128), 128).astype(jnp.int32)
    x = _make_sc_gather(n)(query_wemb, idx)

    ct = jnp.transpose(centroids_k, (0, 2, 1))
    dr = jnp.arange(D)
    w = (jnp.zeros((D, SUB, D, K), jnp.float32)
         .at[dr, :, dr, :].set(ct).reshape(EMB, DK))
    wt = (jnp.zeros((D, K, D, SUB), jnp.float32)
          .at[dr, :, dr, :].set(centroids_k).reshape(DK, EMB))
    wt_hi = wt.astype(jnp.bfloat16)
    wt_lo = (wt - wt_hi.astype(jnp.float32)).astype(jnp.bfloat16)

    bn = 512
    grid = (n // bn,)
    sums, sqs = pl.pallas_call(
        _stats_kernel,
        grid=grid,
        in_specs=[
            pl.BlockSpec((bn, EMB), lambda i: (i, 0)),
            pl.BlockSpec((EMB, DK), lambda i: (0, 0)),
        ],
        out_specs=[
            pl.BlockSpec((1, K), lambda i: (0, 0)),
            pl.BlockSpec((1, K), lambda i: (0, 0)),
        ],
        out_shape=[
            jax.ShapeDtypeStruct((1, K), jnp.float32),
            jax.ShapeDtypeStruct((1, K), jnp.float32),
        ],
    )(x, w)

    out = pl.pallas_call(
        functools.partial(_argmax_kernel, n),
        grid=grid,
        in_specs=[
            pl.BlockSpec((bn, EMB), lambda i: (i, 0)),
            pl.BlockSpec((EMB, DK), lambda i: (0, 0)),
            pl.BlockSpec((DK, EMB), lambda i: (0, 0)),
            pl.BlockSpec((DK, EMB), lambda i: (0, 0)),
            pl.BlockSpec((1, K), lambda i: (0, 0)),
            pl.BlockSpec((1, K), lambda i: (0, 0)),
        ],
        out_specs=pl.BlockSpec((bn, EMB), lambda i: (i, 0)),
        out_shape=jax.ShapeDtypeStruct((n, EMB), jnp.float32),
    )(x, w, wt_hi, wt_lo, sums, sqs)

    return out.reshape(b, h, EMB)


# speedup vs baseline: 2.1247x; 1.2341x over previous
import functools

import jax
import jax.numpy as jnp
from jax import lax
from jax.experimental import pallas as pl
from jax.experimental.pallas import tpu as pltpu
from jax.experimental.pallas import tpu_sc as plsc

K = 512
D = 8
EMB = 64
SUB = EMB // D
DK = D * K

HIGH = lax.Precision.HIGHEST


def _make_sc_gather(n_rows: int):
    info = plsc.get_sparse_core_info()
    nw = info.num_cores * info.num_subcores
    ch = 128
    assert n_rows % (nw * ch) == 0
    nch = n_rows // (nw * ch)
    mesh = plsc.VectorSubcoreMesh(core_axis_name="c", subcore_axis_name="s")

    @functools.partial(
        pl.kernel,
        mesh=mesh,
        out_type=jax.ShapeDtypeStruct((n_rows, EMB), jnp.float32),
        scratch_types=[
            pltpu.VMEM((nch, ch), jnp.int32),
            pltpu.VMEM((ch, EMB), jnp.float32),
            pltpu.VMEM((ch, EMB), jnp.float32),
            pltpu.SemaphoreType.DMA,
            pltpu.SemaphoreType.DMA,
        ],
        compiler_params=pltpu.CompilerParams(use_tc_tiling_on_sc=False),
    )
    def sc_gather(table_hbm, idx_hbm, out_hbm, idx_v, buf0, buf1, sem0, sem1):
        wid = lax.axis_index("s") * info.num_cores + lax.axis_index("c")
        row0 = wid * nch
        pltpu.sync_copy(idx_hbm.at[wid], idx_v)
        bufs = (buf0, buf1)
        sems = (sem0, sem1)
        pltpu.async_copy(table_hbm.at[idx_v.at[0]], bufs[0], sems[0])
        for c in range(nch):
            if c + 1 < nch:
                pltpu.async_copy(
                    table_hbm.at[idx_v.at[c + 1]], bufs[(c + 1) % 2],
                    sems[(c + 1) % 2])
            pltpu.make_async_copy(
                table_hbm.at[idx_v.at[c]], bufs[c % 2], sems[c % 2]).wait()
            pltpu.sync_copy(bufs[c % 2],
                            out_hbm.at[pl.ds((row0 + c) * ch, ch)])

    return sc_gather


def _make_sc_lookup(n_out: int):
    info = plsc.get_sparse_core_info()
    nw = info.num_cores * info.num_subcores
    ch = 128
    assert n_out % (nw * ch * 2) == 0
    nch = n_out // (nw * ch)
    mesh = plsc.VectorSubcoreMesh(core_axis_name="c", subcore_axis_name="s")

    @functools.partial(
        pl.kernel,
        mesh=mesh,
        out_type=jax.ShapeDtypeStruct((n_out, SUB), jnp.float32),
        scratch_types=[
            pltpu.VMEM((nch, ch), jnp.int32),
            pltpu.VMEM((ch, SUB), jnp.float32),
            pltpu.VMEM((ch, SUB), jnp.float32),
            pltpu.SemaphoreType.DMA,
            pltpu.SemaphoreType.DMA,
        ],
        compiler_params=pltpu.CompilerParams(use_tc_tiling_on_sc=False),
    )
    def sc_lookup(cent_hbm, nbr_hbm, out_hbm, idx_v, buf0, buf1, sem0, sem1):
        wid = lax.axis_index("s") * info.num_cores + lax.axis_index("c")
        row0 = wid * nch * ch
        pltpu.sync_copy(nbr_hbm.at[wid], idx_v)

        def body(i, carry):
            c0 = 2 * i
            c1 = c0 + 1
            pltpu.async_copy(cent_hbm.at[idx_v.at[c0]], buf0, sem0)
            pltpu.async_copy(cent_hbm.at[idx_v.at[c1]], buf1, sem1)
            pltpu.make_async_copy(
                cent_hbm.at[idx_v.at[c0]], buf0, sem0).wait()
            pltpu.sync_copy(buf0, out_hbm.at[pl.ds(row0 + c0 * ch, ch)])
            pltpu.make_async_copy(
                cent_hbm.at[idx_v.at[c1]], buf1, sem1).wait()
            pltpu.sync_copy(buf1, out_hbm.at[pl.ds(row0 + c1 * ch, ch)])
            return carry

        lax.fori_loop(0, nch // 2, body, 0)

    return sc_lookup


def _stats_kernel(x_ref, w_ref, sum_ref, sq_ref):
    @pl.when(pl.program_id(0) == 0)
    def _init():
        sum_ref[...] = jnp.zeros_like(sum_ref)
        sq_ref[...] = jnp.zeros_like(sq_ref)

    x = x_ref[...]
    dot = lax.dot_general(x, w_ref[...], (((1,), (0,)), ((), ())),
                          preferred_element_type=jnp.float32)
    c2 = jnp.sum(w_ref[...] * w_ref[...], axis=0, keepdims=True)
    xsq = x * x
    s_acc = jnp.zeros((1, K), jnp.float32)
    q_acc = jnp.zeros((1, K), jnp.float32)
    for d in range(D):
        a = jnp.sum(xsq[:, d * SUB:(d + 1) * SUB], axis=1, keepdims=True)
        r = (-a + 2.0 * dot[:, d * K:(d + 1) * K]) - c2[:, d * K:(d + 1) * K]
        s_acc = s_acc + jnp.sum(r, axis=0, keepdims=True)
        q_acc = q_acc + jnp.sum(r * r, axis=0, keepdims=True)
    sum_ref[...] += s_acc
    sq_ref[...] += q_acc


def _argmax_kernel(n_total, x_ref, w_ref, sum_ref, sq_ref, out_ref):
    nd = jnp.float32(n_total * D)
    mean = sum_ref[...] / nd
    var = sq_ref[...] / nd - (sum_ref[...] / nd) * (sum_ref[...] / nd)
    inv = 1.0 / jnp.sqrt(var + 1e-3)
    x = x_ref[...]
    bn = x.shape[0]
    dot = lax.dot_general(x, w_ref[...], (((1,), (0,)), ((), ())),
                          preferred_element_type=jnp.float32)
    c2 = jnp.sum(w_ref[...] * w_ref[...], axis=0, keepdims=True)
    xsq = x * x
    iota = lax.broadcasted_iota(jnp.int32, (bn, K), 1)
    ohs = []
    for d in range(D):
        a = jnp.sum(xsq[:, d * SUB:(d + 1) * SUB], axis=1, keepdims=True)
        r = (-a + 2.0 * dot[:, d * K:(d + 1) * K]) - c2[:, d * K:(d + 1) * K]
        rn = (r - mean) * inv
        m = jnp.max(rn, axis=1, keepdims=True)
        code = jnp.min(jnp.where(rn == m, iota, K), axis=1, keepdims=True)
        ohs.append(code + d * K)
    out_ref[...] = jnp.concatenate(ohs, axis=1)


def kernel(indices, query_wemb, centroids_k):
    b, h = indices.shape
    n = b * h
    idx = indices.reshape(32, n // (32 * 128), 128).astype(jnp.int32)
    x = _make_sc_gather(n)(query_wemb, idx)

    ct = jnp.transpose(centroids_k, (0, 2, 1))
    dr = jnp.arange(D)
    w = (jnp.zeros((D, SUB, D, K), jnp.float32)
         .at[dr, :, dr, :].set(ct).reshape(EMB, DK))
    cent_flat = centroids_k.reshape(DK, SUB)

    bn = 512
    grid = (n // bn,)
    sums, sqs = pl.pallas_call(
        _stats_kernel,
        grid=grid,
        in_specs=[
            pl.BlockSpec((bn, EMB), lambda i: (i, 0)),
            pl.BlockSpec((EMB, DK), lambda i: (0, 0)),
        ],
        out_specs=[
            pl.BlockSpec((1, K), lambda i: (0, 0)),
            pl.BlockSpec((1, K), lambda i: (0, 0)),
        ],
        out_shape=[
            jax.ShapeDtypeStruct((1, K), jnp.float32),
            jax.ShapeDtypeStruct((1, K), jnp.float32),
        ],
    )(x, w)

    nbr = pl.pallas_call(
        functools.partial(_argmax_kernel, n),
        grid=grid,
        in_specs=[
            pl.BlockSpec((bn, EMB), lambda i: (i, 0)),
            pl.BlockSpec((EMB, DK), lambda i: (0, 0)),
            pl.BlockSpec((1, K), lambda i: (0, 0)),
            pl.BlockSpec((1, K), lambda i: (0, 0)),
        ],
        out_specs=pl.BlockSpec((bn, D), lambda i: (i, 0)),
        out_shape=jax.ShapeDtypeStruct((n, D), jnp.int32),
    )(x, w, sums, sqs)

    n_out = n * D
    nbr3 = nbr.reshape(32, n_out // (32 * 128), 128)
    sel = _make_sc_lookup(n_out)(cent_flat, nbr3)
    return sel.reshape(b, h, EMB)


# speedup vs baseline: 2.2831x; 1.0745x over previous
import functools

import jax
import jax.numpy as jnp
from jax import lax
from jax.experimental import pallas as pl
from jax.experimental.pallas import tpu as pltpu
from jax.experimental.pallas import tpu_sc as plsc

K = 512
D = 8
EMB = 64
SUB = EMB // D
DK = D * K

HIGH = lax.Precision.HIGHEST


def _make_sc_gather(n_rows: int):
    info = plsc.get_sparse_core_info()
    nw = info.num_cores * info.num_subcores
    ch = 128
    assert n_rows % (nw * ch) == 0
    nch = n_rows // (nw * ch)
    mesh = plsc.VectorSubcoreMesh(core_axis_name="c", subcore_axis_name="s")

    @functools.partial(
        pl.kernel,
        mesh=mesh,
        out_type=jax.ShapeDtypeStruct((n_rows, EMB), jnp.float32),
        scratch_types=[
            pltpu.VMEM((nch, ch), jnp.int32),
            pltpu.VMEM((ch, EMB), jnp.float32),
            pltpu.VMEM((ch, EMB), jnp.float32),
            pltpu.SemaphoreType.DMA,
            pltpu.SemaphoreType.DMA,
        ],
        compiler_params=pltpu.CompilerParams(use_tc_tiling_on_sc=False),
    )
    def sc_gather(table_hbm, idx_hbm, out_hbm, idx_v, buf0, buf1, sem0, sem1):
        wid = lax.axis_index("s") * info.num_cores + lax.axis_index("c")
        row0 = wid * nch
        pltpu.sync_copy(idx_hbm.at[wid], idx_v)
        bufs = (buf0, buf1)
        sems = (sem0, sem1)
        pltpu.async_copy(table_hbm.at[idx_v.at[0]], bufs[0], sems[0])
        for c in range(nch):
            if c + 1 < nch:
                pltpu.async_copy(
                    table_hbm.at[idx_v.at[c + 1]], bufs[(c + 1) % 2],
                    sems[(c + 1) % 2])
            pltpu.make_async_copy(
                table_hbm.at[idx_v.at[c]], bufs[c % 2], sems[c % 2]).wait()
            pltpu.sync_copy(bufs[c % 2],
                            out_hbm.at[pl.ds((row0 + c) * ch, ch)])

    return sc_gather


def _make_sc_lookup(n_out: int):
    info = plsc.get_sparse_core_info()
    nw = info.num_cores * info.num_subcores
    ch = 128
    assert n_out % (nw * ch * 2) == 0
    nch = n_out // (nw * ch)
    mesh = plsc.VectorSubcoreMesh(core_axis_name="c", subcore_axis_name="s")

    @functools.partial(
        pl.kernel,
        mesh=mesh,
        out_type=jax.ShapeDtypeStruct((n_out, SUB), jnp.float32),
        scratch_types=[
            pltpu.VMEM((nch, ch), jnp.int32),
            pltpu.VMEM((ch, SUB), jnp.float32),
            pltpu.VMEM((ch, SUB), jnp.float32),
            pltpu.SemaphoreType.DMA,
            pltpu.SemaphoreType.DMA,
        ],
        compiler_params=pltpu.CompilerParams(use_tc_tiling_on_sc=False),
    )
    def sc_lookup(cent_hbm, nbr_hbm, out_hbm, idx_v, buf0, buf1, sem0, sem1):
        wid = lax.axis_index("s") * info.num_cores + lax.axis_index("c")
        row0 = wid * nch * ch
        pltpu.sync_copy(nbr_hbm.at[wid], idx_v)

        def body(i, carry):
            c0 = 2 * i
            c1 = c0 + 1
            pltpu.async_copy(cent_hbm.at[idx_v.at[c0]], buf0, sem0)
            pltpu.async_copy(cent_hbm.at[idx_v.at[c1]], buf1, sem1)
            pltpu.make_async_copy(
                cent_hbm.at[idx_v.at[c0]], buf0, sem0).wait()
            pltpu.sync_copy(buf0, out_hbm.at[pl.ds(row0 + c0 * ch, ch)])
            pltpu.make_async_copy(
                cent_hbm.at[idx_v.at[c1]], buf1, sem1).wait()
            pltpu.sync_copy(buf1, out_hbm.at[pl.ds(row0 + c1 * ch, ch)])
            return carry

        lax.fori_loop(0, nch // 2, body, 0)

    return sc_lookup


def _stats_kernel(x_ref, w_ref, sum_ref, sq_ref):
    @pl.when(pl.program_id(0) == 0)
    def _init():
        sum_ref[...] = jnp.zeros_like(sum_ref)
        sq_ref[...] = jnp.zeros_like(sq_ref)

    x = x_ref[...]
    dot = lax.dot_general(x, w_ref[...], (((1,), (0,)), ((), ())),
                          preferred_element_type=jnp.float32)
    c2 = jnp.sum(w_ref[...] * w_ref[...], axis=0, keepdims=True)
    xsq = x * x
    s_acc = jnp.zeros((1, K), jnp.float32)
    q_acc = jnp.zeros((1, K), jnp.float32)
    for d in range(D):
        a = jnp.sum(xsq[:, d * SUB:(d + 1) * SUB], axis=1, keepdims=True)
        r = (-a + 2.0 * dot[:, d * K:(d + 1) * K]) - c2[:, d * K:(d + 1) * K]
        s_acc = s_acc + jnp.sum(r, axis=0, keepdims=True)
        q_acc = q_acc + jnp.sum(r * r, axis=0, keepdims=True)
    sum_ref[...] += s_acc
    sq_ref[...] += q_acc


def _argmax_kernel(n_total, x_ref, w_ref, sum_ref, sq_ref, out_ref):
    nd = jnp.float32(n_total * D)
    mean = sum_ref[...] / nd
    var = sq_ref[...] / nd - (sum_ref[...] / nd) * (sum_ref[...] / nd)
    inv = 1.0 / jnp.sqrt(var + 1e-3)
    x = x_ref[...]
    bn = x.shape[0]
    dot = lax.dot_general(x, w_ref[...], (((1,), (0,)), ((), ())),
                          preferred_element_type=jnp.float32)
    c2 = jnp.sum(w_ref[...] * w_ref[...], axis=0, keepdims=True)
    xsq = x * x
    iota = lax.broadcasted_iota(jnp.int32, (bn, K), 1)
    ohs = []
    for d in range(D):
        a = jnp.sum(xsq[:, d * SUB:(d + 1) * SUB], axis=1, keepdims=True)
        r = (-a + 2.0 * dot[:, d * K:(d + 1) * K]) - c2[:, d * K:(d + 1) * K]
        rn = (r - mean) * inv
        m = jnp.max(rn, axis=1, keepdims=True)
        code = jnp.min(jnp.where(rn == m, iota, K), axis=1, keepdims=True)
        ohs.append(code + d * K)
    out_ref[...] = jnp.concatenate(ohs, axis=1)


def kernel(indices, query_wemb, centroids_k):
    b, h = indices.shape
    n = b * h
    idx = indices.reshape(32, n // (32 * 128), 128).astype(jnp.int32)
    x = _make_sc_gather(n)(query_wemb, idx)

    ct = jnp.transpose(centroids_k, (0, 2, 1))
    dr = jnp.arange(D)
    w = (jnp.zeros((D, SUB, D, K), jnp.float32)
         .at[dr, :, dr, :].set(ct).reshape(EMB, DK))
    cent_flat = centroids_k.reshape(DK, SUB)

    bn = 1024
    grid = (n // bn,)
    sums, sqs = pl.pallas_call(
        _stats_kernel,
        grid=grid,
        in_specs=[
            pl.BlockSpec((bn, EMB), lambda i: (i, 0)),
            pl.BlockSpec((EMB, DK), lambda i: (0, 0)),
        ],
        out_specs=[
            pl.BlockSpec((1, K), lambda i: (0, 0)),
            pl.BlockSpec((1, K), lambda i: (0, 0)),
        ],
        out_shape=[
            jax.ShapeDtypeStruct((1, K), jnp.float32),
            jax.ShapeDtypeStruct((1, K), jnp.float32),
        ],
    )(x, w)

    nbr = pl.pallas_call(
        functools.partial(_argmax_kernel, n),
        grid=grid,
        in_specs=[
            pl.BlockSpec((bn, EMB), lambda i: (i, 0)),
            pl.BlockSpec((EMB, DK), lambda i: (0, 0)),
            pl.BlockSpec((1, K), lambda i: (0, 0)),
            pl.BlockSpec((1, K), lambda i: (0, 0)),
        ],
        out_specs=pl.BlockSpec((bn, D), lambda i: (i, 0)),
        out_shape=jax.ShapeDtypeStruct((n, D), jnp.int32),
    )(x, w, sums, sqs)

    n_out = n * D
    nbr3 = nbr.reshape(32, n_out // (32 * 128), 128)
    sel = _make_sc_lookup(n_out)(cent_flat, nbr3)
    return sel.reshape(b, h, EMB)


# speedup vs baseline: 2.3322x; 1.0215x over previous
import functools

import jax
import jax.numpy as jnp
from jax import lax
from jax.experimental import pallas as pl
from jax.experimental.pallas import tpu as pltpu
from jax.experimental.pallas import tpu_sc as plsc

K = 512
D = 8
EMB = 64
SUB = EMB // D
DK = D * K

HIGH = lax.Precision.HIGHEST


def _make_sc_gather(n_rows: int):
    info = plsc.get_sparse_core_info()
    nw = info.num_cores * info.num_subcores
    ch = 128
    assert n_rows % (nw * ch) == 0
    nch = n_rows // (nw * ch)
    mesh = plsc.VectorSubcoreMesh(core_axis_name="c", subcore_axis_name="s")

    @functools.partial(
        pl.kernel,
        mesh=mesh,
        out_type=jax.ShapeDtypeStruct((n_rows, EMB), jnp.float32),
        scratch_types=[
            pltpu.VMEM((nch, ch), jnp.int32),
            pltpu.VMEM((ch, EMB), jnp.float32),
            pltpu.VMEM((ch, EMB), jnp.float32),
            pltpu.SemaphoreType.DMA,
            pltpu.SemaphoreType.DMA,
        ],
        compiler_params=pltpu.CompilerParams(use_tc_tiling_on_sc=False),
    )
    def sc_gather(table_hbm, idx_hbm, out_hbm, idx_v, buf0, buf1, sem0, sem1):
        wid = lax.axis_index("s") * info.num_cores + lax.axis_index("c")
        row0 = wid * nch
        pltpu.sync_copy(idx_hbm.at[wid], idx_v)
        bufs = (buf0, buf1)
        sems = (sem0, sem1)
        pltpu.async_copy(table_hbm.at[idx_v.at[0]], bufs[0], sems[0])
        for c in range(nch):
            if c + 1 < nch:
                pltpu.async_copy(
                    table_hbm.at[idx_v.at[c + 1]], bufs[(c + 1) % 2],
                    sems[(c + 1) % 2])
            pltpu.make_async_copy(
                table_hbm.at[idx_v.at[c]], bufs[c % 2], sems[c % 2]).wait()
            pltpu.sync_copy(bufs[c % 2],
                            out_hbm.at[pl.ds((row0 + c) * ch, ch)])

    return sc_gather


def _make_sc_lookup(n_out: int):
    info = plsc.get_sparse_core_info()
    nw = info.num_cores * info.num_subcores
    ch = 128
    assert n_out % (nw * ch * 2) == 0
    nch = n_out // (nw * ch)
    mesh = plsc.VectorSubcoreMesh(core_axis_name="c", subcore_axis_name="s")

    @functools.partial(
        pl.kernel,
        mesh=mesh,
        out_type=jax.ShapeDtypeStruct((n_out, SUB), jnp.float32),
        scratch_types=[
            pltpu.VMEM((nch, ch), jnp.int32),
            pltpu.VMEM((ch, SUB), jnp.float32),
            pltpu.VMEM((ch, SUB), jnp.float32),
            pltpu.SemaphoreType.DMA,
            pltpu.SemaphoreType.DMA,
        ],
        compiler_params=pltpu.CompilerParams(use_tc_tiling_on_sc=False),
    )
    def sc_lookup(cent_hbm, nbr_hbm, out_hbm, idx_v, buf0, buf1, sem0, sem1):
        wid = lax.axis_index("s") * info.num_cores + lax.axis_index("c")
        row0 = wid * nch * ch
        pltpu.sync_copy(nbr_hbm.at[wid], idx_v)

        def body(i, carry):
            c0 = 2 * i
            c1 = c0 + 1
            pltpu.async_copy(cent_hbm.at[idx_v.at[c0]], buf0, sem0)
            pltpu.async_copy(cent_hbm.at[idx_v.at[c1]], buf1, sem1)
            pltpu.make_async_copy(
                cent_hbm.at[idx_v.at[c0]], buf0, sem0).wait()
            pltpu.sync_copy(buf0, out_hbm.at[pl.ds(row0 + c0 * ch, ch)])
            pltpu.make_async_copy(
                cent_hbm.at[idx_v.at[c1]], buf1, sem1).wait()
            pltpu.sync_copy(buf1, out_hbm.at[pl.ds(row0 + c1 * ch, ch)])
            return carry

        lax.fori_loop(0, nch // 2, body, 0)

    return sc_lookup


def _stats_kernel(x_ref, w_ref, sum_ref, sq_ref):
    @pl.when(pl.program_id(0) == 0)
    def _init():
        sum_ref[...] = jnp.zeros_like(sum_ref)
        sq_ref[...] = jnp.zeros_like(sq_ref)

    x = x_ref[...]
    dot = lax.dot_general(x, w_ref[...], (((1,), (0,)), ((), ())),
                          preferred_element_type=jnp.float32)
    c2 = jnp.sum(w_ref[...] * w_ref[...], axis=0, keepdims=True)
    xsq = x * x
    s_acc = jnp.zeros((1, K), jnp.float32)
    q_acc = jnp.zeros((1, K), jnp.float32)
    for d in range(D):
        a = jnp.sum(xsq[:, d * SUB:(d + 1) * SUB], axis=1, keepdims=True)
        r = (-a + 2.0 * dot[:, d * K:(d + 1) * K]) - c2[:, d * K:(d + 1) * K]
        s_acc = s_acc + jnp.sum(r, axis=0, keepdims=True)
        q_acc = q_acc + jnp.sum(r * r, axis=0, keepdims=True)
    sum_ref[...] += s_acc
    sq_ref[...] += q_acc


def _argmax_kernel(n_total, x_ref, w_ref, sum_ref, sq_ref, out_ref):
    nd = jnp.float32(n_total * D)
    mean = sum_ref[...] / nd
    var = sq_ref[...] / nd - (sum_ref[...] / nd) * (sum_ref[...] / nd)
    inv = 1.0 / jnp.sqrt(var + 1e-3)
    x = x_ref[...]
    bn = x.shape[0]
    dot = lax.dot_general(x, w_ref[...], (((1,), (0,)), ((), ())),
                          preferred_element_type=jnp.float32)
    c2 = jnp.sum(w_ref[...] * w_ref[...], axis=0, keepdims=True)
    xsq = x * x
    iota = lax.broadcasted_iota(jnp.int32, (bn, K), 1)
    ohs = []
    for d in range(D):
        a = jnp.sum(xsq[:, d * SUB:(d + 1) * SUB], axis=1, keepdims=True)
        r = (-a + 2.0 * dot[:, d * K:(d + 1) * K]) - c2[:, d * K:(d + 1) * K]
        rn = (r - mean) * inv
        m = jnp.max(rn, axis=1, keepdims=True)
        code = jnp.min(jnp.where(rn == m, iota, K), axis=1, keepdims=True)
        ohs.append(code + d * K)
    out_ref[...] = jnp.concatenate(ohs, axis=1)


def kernel(indices, query_wemb, centroids_k):
    b, h = indices.shape
    n = b * h
    idx = indices.reshape(32, n // (32 * 128), 128).astype(jnp.int32)
    x = _make_sc_gather(n)(query_wemb, idx)

    ct = jnp.transpose(centroids_k, (0, 2, 1))
    dr = jnp.arange(D)
    w = (jnp.zeros((D, SUB, D, K), jnp.float32)
         .at[dr, :, dr, :].set(ct).reshape(EMB, DK))
    cent_flat = centroids_k.reshape(DK, SUB)

    bn = 2048
    grid = (n // bn,)
    sums, sqs = pl.pallas_call(
        _stats_kernel,
        grid=grid,
        in_specs=[
            pl.BlockSpec((bn, EMB), lambda i: (i, 0)),
            pl.BlockSpec((EMB, DK), lambda i: (0, 0)),
        ],
        out_specs=[
            pl.BlockSpec((1, K), lambda i: (0, 0)),
            pl.BlockSpec((1, K), lambda i: (0, 0)),
        ],
        out_shape=[
            jax.ShapeDtypeStruct((1, K), jnp.float32),
            jax.ShapeDtypeStruct((1, K), jnp.float32),
        ],
    )(x, w)

    nbr = pl.pallas_call(
        functools.partial(_argmax_kernel, n),
        grid=grid,
        in_specs=[
            pl.BlockSpec((bn, EMB), lambda i: (i, 0)),
            pl.BlockSpec((EMB, DK), lambda i: (0, 0)),
            pl.BlockSpec((1, K), lambda i: (0, 0)),
            pl.BlockSpec((1, K), lambda i: (0, 0)),
        ],
        out_specs=pl.BlockSpec((bn, D), lambda i: (i, 0)),
        out_shape=jax.ShapeDtypeStruct((n, D), jnp.int32),
    )(x, w, sums, sqs)

    n_out = n * D
    nbr3 = nbr.reshape(32, n_out // (32 * 128), 128)
    sel = _make_sc_lookup(n_out)(cent_flat, nbr3)
    return sel.reshape(b, h, EMB)


# speedup vs baseline: 2.3820x; 1.0213x over previous
import functools

import jax
import jax.numpy as jnp
from jax import lax
from jax.experimental import pallas as pl
from jax.experimental.pallas import tpu as pltpu
from jax.experimental.pallas import tpu_sc as plsc

K = 512
D = 8
EMB = 64
SUB = EMB // D
DK = D * K

HIGH = lax.Precision.HIGHEST


def _make_sc_gather(n_rows: int):
    info = plsc.get_sparse_core_info()
    nw = info.num_cores * info.num_subcores
    ch = 128
    assert n_rows % (nw * ch) == 0
    nch = n_rows // (nw * ch)
    mesh = plsc.VectorSubcoreMesh(core_axis_name="c", subcore_axis_name="s")

    @functools.partial(
        pl.kernel,
        mesh=mesh,
        out_type=jax.ShapeDtypeStruct((n_rows, EMB), jnp.float32),
        scratch_types=[
            pltpu.VMEM((nch, ch), jnp.int32),
            pltpu.VMEM((ch, EMB), jnp.float32),
            pltpu.VMEM((ch, EMB), jnp.float32),
            pltpu.SemaphoreType.DMA,
            pltpu.SemaphoreType.DMA,
        ],
        compiler_params=pltpu.CompilerParams(use_tc_tiling_on_sc=False),
    )
    def sc_gather(table_hbm, idx_hbm, out_hbm, idx_v, buf0, buf1, sem0, sem1):
        wid = lax.axis_index("s") * info.num_cores + lax.axis_index("c")
        row0 = wid * nch
        pltpu.sync_copy(idx_hbm.at[wid], idx_v)
        bufs = (buf0, buf1)
        sems = (sem0, sem1)
        pltpu.async_copy(table_hbm.at[idx_v.at[0]], bufs[0], sems[0])
        for c in range(nch):
            if c + 1 < nch:
                pltpu.async_copy(
                    table_hbm.at[idx_v.at[c + 1]], bufs[(c + 1) % 2],
                    sems[(c + 1) % 2])
            pltpu.make_async_copy(
                table_hbm.at[idx_v.at[c]], bufs[c % 2], sems[c % 2]).wait()
            pltpu.sync_copy(bufs[c % 2],
                            out_hbm.at[pl.ds((row0 + c) * ch, ch)])

    return sc_gather


def _make_sc_lookup(n_out: int):
    info = plsc.get_sparse_core_info()
    nw = info.num_cores * info.num_subcores
    ch = 128
    assert n_out % (nw * ch * 2) == 0
    nch = n_out // (nw * ch)
    mesh = plsc.VectorSubcoreMesh(core_axis_name="c", subcore_axis_name="s")

    @functools.partial(
        pl.kernel,
        mesh=mesh,
        out_type=jax.ShapeDtypeStruct((n_out, SUB), jnp.float32),
        scratch_types=[
            pltpu.VMEM((nch, ch), jnp.int32),
            pltpu.VMEM((ch, SUB), jnp.float32),
            pltpu.VMEM((ch, SUB), jnp.float32),
            pltpu.SemaphoreType.DMA,
            pltpu.SemaphoreType.DMA,
        ],
        compiler_params=pltpu.CompilerParams(use_tc_tiling_on_sc=False),
    )
    def sc_lookup(cent_hbm, nbr_hbm, out_hbm, idx_v, buf0, buf1, sem0, sem1):
        wid = lax.axis_index("s") * info.num_cores + lax.axis_index("c")
        row0 = wid * nch * ch
        pltpu.sync_copy(nbr_hbm.at[wid], idx_v)

        def body(i, carry):
            c0 = 2 * i
            c1 = c0 + 1
            pltpu.async_copy(cent_hbm.at[idx_v.at[c0]], buf0, sem0)
            pltpu.async_copy(cent_hbm.at[idx_v.at[c1]], buf1, sem1)
            pltpu.make_async_copy(
                cent_hbm.at[idx_v.at[c0]], buf0, sem0).wait()
            pltpu.sync_copy(buf0, out_hbm.at[pl.ds(row0 + c0 * ch, ch)])
            pltpu.make_async_copy(
                cent_hbm.at[idx_v.at[c1]], buf1, sem1).wait()
            pltpu.sync_copy(buf1, out_hbm.at[pl.ds(row0 + c1 * ch, ch)])
            return carry

        lax.fori_loop(0, nch // 2, body, 0)

    return sc_lookup


def _stats_kernel(x_ref, w_ref, sum_ref, sq_ref):
    @pl.when(pl.program_id(0) == 0)
    def _init():
        sum_ref[...] = jnp.zeros_like(sum_ref)
        sq_ref[...] = jnp.zeros_like(sq_ref)

    x = x_ref[...]
    dot = lax.dot_general(x, w_ref[...], (((1,), (0,)), ((), ())),
                          preferred_element_type=jnp.float32)
    c2 = jnp.sum(w_ref[...] * w_ref[...], axis=0, keepdims=True)
    xsq = x * x
    s_acc = jnp.zeros((1, K), jnp.float32)
    q_acc = jnp.zeros((1, K), jnp.float32)
    for d in range(D):
        a = jnp.sum(xsq[:, d * SUB:(d + 1) * SUB], axis=1, keepdims=True)
        r = (-a + 2.0 * dot[:, d * K:(d + 1) * K]) - c2[:, d * K:(d + 1) * K]
        s_acc = s_acc + jnp.sum(r, axis=0, keepdims=True)
        q_acc = q_acc + jnp.sum(r * r, axis=0, keepdims=True)
    sum_ref[...] += s_acc
    sq_ref[...] += q_acc


def _argmax_kernel(n_total, x_ref, w_ref, sum_ref, sq_ref, out_ref):
    nd = jnp.float32(n_total * D)
    mean = sum_ref[...] / nd
    var = sq_ref[...] / nd - (sum_ref[...] / nd) * (sum_ref[...] / nd)
    inv = 1.0 / jnp.sqrt(var + 1e-3)
    x = x_ref[...]
    bn = x.shape[0]
    dot = lax.dot_general(x, w_ref[...], (((1,), (0,)), ((), ())),
                          preferred_element_type=jnp.float32)
    c2 = jnp.sum(w_ref[...] * w_ref[...], axis=0, keepdims=True)
    xsq = x * x
    iota = lax.broadcasted_iota(jnp.int32, (bn, K), 1)
    ohs = []
    for d in range(D):
        a = jnp.sum(xsq[:, d * SUB:(d + 1) * SUB], axis=1, keepdims=True)
        c2m = c2[:, d * K:(d + 1) * K] + mean
        rn = ((2.0 * dot[:, d * K:(d + 1) * K] - a) - c2m) * inv
        m = jnp.max(rn, axis=1, keepdims=True)
        code = jnp.min(jnp.where(rn == m, iota, K), axis=1, keepdims=True)
        ohs.append(code + d * K)
    out_ref[...] = jnp.concatenate(ohs, axis=1)


def kernel(indices, query_wemb, centroids_k):
    b, h = indices.shape
    n = b * h
    idx = indices.reshape(32, n // (32 * 128), 128).astype(jnp.int32)
    x = _make_sc_gather(n)(query_wemb, idx)

    ct = jnp.transpose(centroids_k, (0, 2, 1))
    dr = jnp.arange(D)
    w = (jnp.zeros((D, SUB, D, K), jnp.float32)
         .at[dr, :, dr, :].set(ct).reshape(EMB, DK))
    cent_flat = centroids_k.reshape(DK, SUB)

    bn = 2048
    grid = (n // bn,)
    sums, sqs = pl.pallas_call(
        _stats_kernel,
        grid=grid,
        in_specs=[
            pl.BlockSpec((bn, EMB), lambda i: (i, 0)),
            pl.BlockSpec((EMB, DK), lambda i: (0, 0)),
        ],
        out_specs=[
            pl.BlockSpec((1, K), lambda i: (0, 0)),
            pl.BlockSpec((1, K), lambda i: (0, 0)),
        ],
        out_shape=[
            jax.ShapeDtypeStruct((1, K), jnp.float32),
            jax.ShapeDtypeStruct((1, K), jnp.float32),
        ],
    )(x, w)

    nbr = pl.pallas_call(
        functools.partial(_argmax_kernel, n),
        grid=grid,
        in_specs=[
            pl.BlockSpec((bn, EMB), lambda i: (i, 0)),
            pl.BlockSpec((EMB, DK), lambda i: (0, 0)),
            pl.BlockSpec((1, K), lambda i: (0, 0)),
            pl.BlockSpec((1, K), lambda i: (0, 0)),
        ],
        out_specs=pl.BlockSpec((bn, D), lambda i: (i, 0)),
        out_shape=jax.ShapeDtypeStruct((n, D), jnp.int32),
    )(x, w, sums, sqs)

    n_out = n * D
    nbr3 = nbr.reshape(32, n_out // (32 * 128), 128)
    sel = _make_sc_lookup(n_out)(cent_flat, nbr3)
    return sel.reshape(b, h, EMB)
